# baseline (speedup 1.0000x reference)
import functools

import jax
import jax.numpy as jnp
from jax.experimental import pallas as pl
from jax.experimental.pallas import tpu as pltpu
from jax.experimental.pallas import tpu_sc as plsc

NUM_TOKENS = 2048
HIDDEN_DIM = 768
FFN_DIM = 512
NUM_EXPERTS = 64

ROUTER_BLK = 256
TILE = 32
PADN = NUM_TOKENS + NUM_EXPERTS * 8
OUT_PAD = NUM_TOKENS + 8

_NW = 32
_RPW = PADN // _NW


def _router_kernel(h_ref, rw_ref, logits_ref, idx_ref, w_ref):
    h = h_ref[...]
    logits = jax.lax.dot_general(
        h, rw_ref[...], (((1,), (1,)), ((), ())),
        preferred_element_type=jnp.float32)
    logits_ref[...] = logits
    m = jnp.max(logits, axis=1, keepdims=True)
    s = jnp.sum(jnp.exp(logits - m), axis=1, keepdims=True)
    idx_ref[...] = jnp.argmax(logits, axis=1, keepdims=True).astype(jnp.int32)
    w_ref[...] = 1.0 / s


def _expert_kernel(off_ref, gidx_ref,
                   h_ref, tws_ref, wg_ref, wu_ref, wd_ref, y_ref, xs_ref):
    e = pl.program_id(0)
    start = pl.multiple_of(off_ref[e], 8)
    end = off_ref[e + 1]
    num_tiles = pl.cdiv(end - start, TILE)

    wg = wg_ref[0].astype(jnp.bfloat16)
    wu = wu_ref[0].astype(jnp.bfloat16)
    wd = wd_ref[0].astype(jnp.bfloat16)

    def tile_body(t, _):
        base = pl.multiple_of(start + t * TILE, 8)
        for r in range(TILE):
            tok = gidx_ref[base + r]
            xs_ref[r:r + 1, :] = h_ref[pl.ds(tok, 1), :]
        x = xs_ref[...].astype(jnp.bfloat16)
        g = jax.lax.dot_general(x, wg, (((1,), (1,)), ((), ())),
                                preferred_element_type=jnp.float32)
        u = jax.lax.dot_general(x, wu, (((1,), (1,)), ((), ())),
                                preferred_element_type=jnp.float32)
        a = g * jax.nn.sigmoid(g) * u
        y = jax.lax.dot_general(a.astype(jnp.bfloat16), wd,
                                (((1,), (1,)), ((), ())),
                                preferred_element_type=jnp.float32)
        y_ref[pl.ds(base, TILE), :] = y * tws_ref[pl.ds(base, TILE), :]
        return 0

    jax.lax.fori_loop(0, num_tiles, tile_body, 0)


def _sc_scatter(y_hbm, sidx_hbm, out_hbm, idx_v, rows_v, sem):
    wid = jax.lax.axis_index("s") * 2 + jax.lax.axis_index("c")
    base = wid * _RPW
    pltpu.sync_copy(sidx_hbm.at[pl.ds(base, _RPW)], idx_v)
    pltpu.sync_copy(y_hbm.at[pl.ds(base, _RPW)], rows_v)
    pltpu.async_copy(rows_v, out_hbm.at[idx_v], sem).wait()


@jax.jit
def kernel(hidden_states, router_weight, w_gate, w_up, w_down):
    logits, idx, topw = pl.pallas_call(
        _router_kernel,
        grid=(NUM_TOKENS // ROUTER_BLK,),
        in_specs=[
            pl.BlockSpec((ROUTER_BLK, HIDDEN_DIM), lambda i: (i, 0)),
            pl.BlockSpec((NUM_EXPERTS, HIDDEN_DIM), lambda i: (0, 0)),
        ],
        out_specs=[
            pl.BlockSpec((ROUTER_BLK, NUM_EXPERTS), lambda i: (i, 0)),
            pl.BlockSpec((ROUTER_BLK, 1), lambda i: (i, 0)),
            pl.BlockSpec((ROUTER_BLK, 1), lambda i: (i, 0)),
        ],
        out_shape=[
            jax.ShapeDtypeStruct((NUM_TOKENS, NUM_EXPERTS), jnp.float32),
            jax.ShapeDtypeStruct((NUM_TOKENS, 1), jnp.int32),
            jax.ShapeDtypeStruct((NUM_TOKENS, 1), jnp.float32),
        ],
    )(hidden_states, router_weight)

    top1 = idx[:, 0]
    counts = jnp.bincount(top1, length=NUM_EXPERTS).astype(jnp.int32)
    pcounts = (counts + 7) & ~7
    poff = jnp.concatenate(
        [jnp.zeros((1,), jnp.int32), jnp.cumsum(pcounts).astype(jnp.int32)])
    off = jnp.concatenate(
        [jnp.zeros((1,), jnp.int32), jnp.cumsum(counts).astype(jnp.int32)])
    order = jnp.argsort(top1).astype(jnp.int32)
    ts = top1[order]
    jp = poff[ts] + jnp.arange(NUM_TOKENS, dtype=jnp.int32) - off[ts]
    gidx = jnp.zeros((PADN,), jnp.int32).at[jp].set(order)
    sidx = jnp.full((PADN,), NUM_TOKENS, jnp.int32).at[jp].set(order)
    tws = topw[gidx]

    y_sorted = pl.pallas_call(
        _expert_kernel,
        grid_spec=pltpu.PrefetchScalarGridSpec(
            num_scalar_prefetch=2,
            grid=(NUM_EXPERTS,),
            in_specs=[
                pl.BlockSpec((NUM_TOKENS, HIDDEN_DIM), lambda e, *_: (0, 0)),
                pl.BlockSpec((PADN, 1), lambda e, *_: (0, 0)),
                pl.BlockSpec((1, FFN_DIM, HIDDEN_DIM), lambda e, *_: (e, 0, 0)),
                pl.BlockSpec((1, FFN_DIM, HIDDEN_DIM), lambda e, *_: (e, 0, 0)),
                pl.BlockSpec((1, HIDDEN_DIM, FFN_DIM), lambda e, *_: (e, 0, 0)),
            ],
            out_specs=pl.BlockSpec((PADN, HIDDEN_DIM), lambda e, *_: (0, 0)),
            scratch_shapes=[pltpu.VMEM((TILE, HIDDEN_DIM), jnp.float32)],
        ),
        out_shape=jax.ShapeDtypeStruct((PADN, HIDDEN_DIM), jnp.float32),
    )(poff, gidx, hidden_states, tws, w_gate, w_up, w_down)

    scatter = pl.kernel(
        _sc_scatter,
        mesh=plsc.VectorSubcoreMesh(core_axis_name="c", subcore_axis_name="s"),
        out_type=jax.ShapeDtypeStruct((OUT_PAD, HIDDEN_DIM), jnp.float32),
        scratch_types=[
            pltpu.VMEM((_RPW,), jnp.int32),
            pltpu.VMEM((_RPW, HIDDEN_DIM), jnp.float32),
            pltpu.SemaphoreType.DMA,
        ],
    )
    outp = scatter(y_sorted, sidx)
    return outp[:NUM_TOKENS], idx, topw, logits


# speedup vs baseline: 1.4375x; 1.4375x over previous
import jax
import jax.numpy as jnp
from jax.experimental import pallas as pl
from jax.experimental.pallas import tpu as pltpu
from jax.experimental.pallas import tpu_sc as plsc

NUM_TOKENS = 2048
HIDDEN_DIM = 768
FFN_DIM = 512
NUM_EXPERTS = 64

TILE = 32
BLK = 128
PADN = NUM_TOKENS + NUM_EXPERTS * 8

_NW = 32
_RPW = NUM_TOKENS // _NW


def _router_kernel(h_ref, rw_ref,
                   logits_ref, idx_ref, w_ref, jp_ref, poff_ref, pc_ref):
    h = h_ref[...]
    logits = jax.lax.dot_general(
        h, rw_ref[...], (((1,), (1,)), ((), ())),
        preferred_element_type=jnp.float32)
    logits_ref[...] = logits
    m = jnp.max(logits, axis=1, keepdims=True)
    s = jnp.sum(jnp.exp(logits - m), axis=1, keepdims=True)
    idx_ref[...] = jnp.argmax(logits, axis=1, keepdims=True).astype(jnp.int32)
    w_ref[...] = 1.0 / s

    oh = jnp.where(logits == m, 1.0, 0.0)
    ecol = jax.lax.broadcasted_iota(jnp.int32, (NUM_EXPERTS, NUM_EXPERTS), 0)
    erow = jax.lax.broadcasted_iota(jnp.int32, (NUM_EXPERTS, NUM_EXPERTS), 1)
    tri_incl = jnp.where(ecol <= erow, 1.0, 0.0)
    tri_strict = jnp.where(ecol < erow, 1.0, 0.0)
    lane_pref = jax.lax.dot_general(oh, tri_incl, (((1,), (0,)), ((), ())),
                                    preferred_element_type=jnp.float32)
    first = oh * jnp.where(lane_pref == 1.0, 1.0, 0.0)

    counts = jnp.sum(first, axis=0, keepdims=True)
    pc = ((counts.astype(jnp.int32) + 7) >> 3) << 3
    pc_ref[...] = pc
    pcf = pc.astype(jnp.float32)
    poff = jax.lax.dot_general(pcf, tri_strict, (((1,), (0,)), ((), ())),
                               preferred_element_type=jnp.float32)
    poff_ref[...] = poff.astype(jnp.int32)

    tcol = jax.lax.broadcasted_iota(jnp.int32, (BLK, BLK), 0)
    trow = jax.lax.broadcasted_iota(jnp.int32, (BLK, BLK), 1)
    tril = jnp.where(trow <= tcol, 1.0, 0.0)
    carry = jnp.zeros((1, NUM_EXPERTS), jnp.float32)
    for b in range(NUM_TOKENS // BLK):
        blk = first[b * BLK:(b + 1) * BLK, :]
        pref = jax.lax.dot_general(tril, blk, (((1,), (0,)), ((), ())),
                                   preferred_element_type=jnp.float32) + carry
        rank = jnp.sum(pref * blk, axis=1, keepdims=True) - 1.0
        base = jnp.sum(poff * blk, axis=1, keepdims=True)
        jp_ref[b * BLK:(b + 1) * BLK, :] = (base + rank).astype(jnp.int32)
        carry = pref[BLK - 1:BLK, :]


def _expert_kernel(poff_ref, pc_ref, gidx_ref,
                   h_ref, tws_ref, wg_ref, wu_ref, wd_ref, y_ref, xs_ref):
    e = pl.program_id(0)
    start = pl.multiple_of(poff_ref[0, e], 8)
    num_tiles = pl.cdiv(pc_ref[0, e], TILE)

    wg = wg_ref[0].astype(jnp.bfloat16)
    wu = wu_ref[0].astype(jnp.bfloat16)
    wd = wd_ref[0].astype(jnp.bfloat16)

    def tile_body(t, _):
        base = pl.multiple_of(start + t * TILE, 8)
        for r in range(TILE):
            tok = gidx_ref[base + r]
            xs_ref[r:r + 1, :] = h_ref[pl.ds(tok, 1), :]
        x = xs_ref[...].astype(jnp.bfloat16)
        g = jax.lax.dot_general(x, wg, (((1,), (1,)), ((), ())),
                                preferred_element_type=jnp.float32)
        u = jax.lax.dot_general(x, wu, (((1,), (1,)), ((), ())),
                                preferred_element_type=jnp.float32)
        a = g * jax.nn.sigmoid(g) * u
        y = jax.lax.dot_general(a.astype(jnp.bfloat16), wd,
                                (((1,), (1,)), ((), ())),
                                preferred_element_type=jnp.float32)
        y_ref[pl.ds(base, TILE), :] = y * tws_ref[pl.ds(base, TILE), :]
        return 0

    jax.lax.fori_loop(0, num_tiles, tile_body, 0)


def _sc_gather(y_hbm, jp_hbm, out_hbm, jp_v, rows_v, sem):
    wid = jax.lax.axis_index("s") * 2 + jax.lax.axis_index("c")
    base = wid * _RPW
    pltpu.sync_copy(jp_hbm.at[pl.ds(base, _RPW)], jp_v)
    pltpu.async_copy(y_hbm.at[jp_v], rows_v, sem).wait()
    pltpu.sync_copy(rows_v, out_hbm.at[pl.ds(base, _RPW)])


@jax.jit
def kernel(hidden_states, router_weight, w_gate, w_up, w_down):
    logits, idx, topw, jp, poff, pc = pl.pallas_call(
        _router_kernel,
        grid=(1,),
        in_specs=[
            pl.BlockSpec((NUM_TOKENS, HIDDEN_DIM), lambda i: (0, 0)),
            pl.BlockSpec((NUM_EXPERTS, HIDDEN_DIM), lambda i: (0, 0)),
        ],
        out_specs=[
            pl.BlockSpec((NUM_TOKENS, NUM_EXPERTS), lambda i: (0, 0)),
            pl.BlockSpec((NUM_TOKENS, 1), lambda i: (0, 0)),
            pl.BlockSpec((NUM_TOKENS, 1), lambda i: (0, 0)),
            pl.BlockSpec((NUM_TOKENS, 1), lambda i: (0, 0)),
            pl.BlockSpec((1, NUM_EXPERTS), lambda i: (0, 0)),
            pl.BlockSpec((1, NUM_EXPERTS), lambda i: (0, 0)),
        ],
        out_shape=[
            jax.ShapeDtypeStruct((NUM_TOKENS, NUM_EXPERTS), jnp.float32),
            jax.ShapeDtypeStruct((NUM_TOKENS, 1), jnp.int32),
            jax.ShapeDtypeStruct((NUM_TOKENS, 1), jnp.float32),
            jax.ShapeDtypeStruct((NUM_TOKENS, 1), jnp.int32),
            jax.ShapeDtypeStruct((1, NUM_EXPERTS), jnp.int32),
            jax.ShapeDtypeStruct((1, NUM_EXPERTS), jnp.int32),
        ],
    )(hidden_states, router_weight)

    jp1 = jp[:, 0]
    gidx = jnp.zeros((PADN,), jnp.int32).at[jp1].set(
        jnp.arange(NUM_TOKENS, dtype=jnp.int32))
    tws = jnp.zeros((PADN, 1), jnp.float32).at[jp1].set(topw)

    y_sorted = pl.pallas_call(
        _expert_kernel,
        grid_spec=pltpu.PrefetchScalarGridSpec(
            num_scalar_prefetch=3,
            grid=(NUM_EXPERTS,),
            in_specs=[
                pl.BlockSpec((NUM_TOKENS, HIDDEN_DIM), lambda e, *_: (0, 0)),
                pl.BlockSpec((PADN, 1), lambda e, *_: (0, 0)),
                pl.BlockSpec((1, FFN_DIM, HIDDEN_DIM), lambda e, *_: (e, 0, 0)),
                pl.BlockSpec((1, FFN_DIM, HIDDEN_DIM), lambda e, *_: (e, 0, 0)),
                pl.BlockSpec((1, HIDDEN_DIM, FFN_DIM), lambda e, *_: (e, 0, 0)),
            ],
            out_specs=pl.BlockSpec((PADN, HIDDEN_DIM), lambda e, *_: (0, 0)),
            scratch_shapes=[pltpu.VMEM((TILE, HIDDEN_DIM), jnp.float32)],
        ),
        out_shape=jax.ShapeDtypeStruct((PADN, HIDDEN_DIM), jnp.float32),
    )(poff, pc, gidx, hidden_states, tws, w_gate, w_up, w_down)

    unpermute = pl.kernel(
        _sc_gather,
        mesh=plsc.VectorSubcoreMesh(core_axis_name="c", subcore_axis_name="s"),
        out_type=jax.ShapeDtypeStruct((NUM_TOKENS, HIDDEN_DIM), jnp.float32),
        scratch_types=[
            pltpu.VMEM((_RPW,), jnp.int32),
            pltpu.VMEM((_RPW, HIDDEN_DIM), jnp.float32),
            pltpu.SemaphoreType.DMA,
        ],
    )
    combined = unpermute(y_sorted, jp1)
    return combined, idx, topw, logits
